# Initial kernel scaffold; baseline (speedup 1.0000x reference)
import jax
import jax.numpy as jnp
from jax.experimental import pallas as pl


def kernel(triplane, triplane_coord_3d, index):
    raise NotImplementedError("write your pallas kernel here")


# speedup vs baseline: 3.2893x; 3.2893x over previous
import functools

import jax
import jax.numpy as jnp
from jax import lax
from jax.experimental import pallas as pl
from jax.experimental.pallas import tpu as pltpu
from jax.experimental.pallas import tpu_sc as plsc

B = 4
C_FEAT = 32
N_PARTS = 16
C = C_FEAT + N_PARTS
H = 512
W = 512
N = 262144
NPTS = 32768
D = 32
NCORNER = 12

NC = 2
NS = 16
NW = NC * NS
LANES = 16
PPT = N // NW
CH = 128
NCH = PPT // CH
K_OUT = B * NPTS
KT = K_OUT // NW
KR = KT // 128


_HB = 8


def _table_body(tp_ref, out_ref):
    x = tp_ref[0]
    row = lax.broadcasted_iota(jnp.int32, (C, D), 0)
    col = lax.broadcasted_iota(jnp.int32, (C, D), 1)
    proj = (jnp.where((row == col + C_FEAT) & (col < N_PARTS), 1.0, 0.0)
            + jnp.where((col == N_PARTS) & (row < C_FEAT), 1.0, 0.0))
    for hb in range(_HB):
        xs = x[:, hb, :]
        out_ref[0, hb] = lax.dot_general(
            xs, proj, (((0,), (0,)), ((), ())),
            preferred_element_type=jnp.float32)


def _pack_table(triplane):
    out = pl.pallas_call(
        _table_body,
        grid=(B, H // _HB),
        in_specs=[pl.BlockSpec((1, C, _HB, W), lambda b, h: (b, 0, h, 0))],
        out_specs=pl.BlockSpec((1, _HB, W, D), lambda b, h: (b, h, 0, 0)),
        out_shape=jax.ShapeDtypeStruct((B, H, W, D), jnp.float32),
    )(triplane)
    return out.reshape(B * H * W, D)


_PB = 2048
_GB = _PB // CH


def _coord_body(ct_ref, bounds_ref, gidx_ref, wts_ref):
    pid = pl.program_id(0)
    shape = (_GB, CH)
    i = (pid * _PB
         + lax.broadcasted_iota(jnp.int32, shape, 0) * CH
         + lax.broadcasted_iota(jnp.int32, shape, 1))
    bid = jnp.zeros(shape, jnp.int32)
    for j in range(B):
        bid = bid + (i >= bounds_ref[0, j]).astype(jnp.int32)
    bid = jnp.minimum(bid, B - 1)
    base = bid * (H * W)
    x = ct_ref[0].reshape(shape)
    y = ct_ref[1].reshape(shape)
    z = ct_ref[2].reshape(shape)
    px = jnp.clip((x + 1.0) * (0.5 * (W - 1.0)), 0.0, W - 1.0)
    py = jnp.clip((y + 1.0) * (0.5 * (H - 1.0)), 0.0, H - 1.0)
    pz = jnp.clip((z + 1.0) * (0.5 * (H - 1.0)), 0.0, H - 1.0)
    gs, ws = [], []
    for (u, v) in ((px, py), (px, pz), (py, pz)):
        bx = jnp.clip(jnp.floor(u), 0.0, W - 2.0)
        by = jnp.clip(jnp.floor(v), 0.0, H - 2.0)
        wx = u - bx
        wy = v - by
        i00 = base + by.astype(jnp.int32) * W + bx.astype(jnp.int32)
        for (dy, dx, wgt) in ((0, 0, (1.0 - wx) * (1.0 - wy)),
                              (0, 1, wx * (1.0 - wy)),
                              (1, 0, (1.0 - wx) * wy),
                              (1, 1, wx * wy)):
            gs.append(i00 + (dy * W + dx))
            ws.append(wgt)
    gidx_ref[...] = jnp.stack(gs, axis=1)
    wts_ref[...] = jnp.stack(ws, axis=1)


def _prep_coords(coords_t, bounds):
    return pl.pallas_call(
        _coord_body,
        grid=(N // _PB,),
        in_specs=[
            pl.BlockSpec((8, _PB), lambda p: (0, p)),
            pl.BlockSpec(memory_space=pltpu.SMEM),
        ],
        out_specs=[
            pl.BlockSpec((_GB, NCORNER, CH), lambda p: (p, 0, 0)),
            pl.BlockSpec((_GB, NCORNER, CH), lambda p: (p, 0, 0)),
        ],
        out_shape=[
            jax.ShapeDtypeStruct((N // CH, NCORNER, CH), jnp.int32),
            jax.ShapeDtypeStruct((N // CH, NCORNER, CH), jnp.float32),
        ],
    )(coords_t, bounds)


def _sc_main_body(table_h, gidx_h, wts_h, s_h, idx_v, wts_v, rows_v, s_v, sem):
    wid = lax.axis_index("s") * NC + lax.axis_index("c")
    chunk0 = wid * NCH

    def chunk_body(cb, carry):
        gcb = chunk0 + cb
        pltpu.sync_copy(gidx_h.at[gcb], idx_v)
        pltpu.sync_copy(wts_h.at[gcb], wts_v)
        copies = [
            pltpu.async_copy(table_h.at[idx_v.at[j]], rows_v.at[j], sem)
            for j in range(NCORNER)
        ]
        for cpy in copies:
            cpy.wait()

        def group_body(g, carry2):
            lvec = lax.iota(jnp.int32, LANES) + g * LANES
            macc = jnp.zeros((LANES,), jnp.float32)
            pacc = [jnp.zeros((LANES,), jnp.float32) for _ in range(N_PARTS)]
            for j in range(NCORNER):
                w = wts_v[j, pl.ds(g * LANES, LANES)]
                jv = jnp.full((LANES,), j, jnp.int32)
                fs = plsc.load_gather(
                    rows_v, [jv, lvec, jnp.full((LANES,), N_PARTS, jnp.int32)])
                macc = macc + w * fs
                for c in range(N_PARTS):
                    pc = plsc.load_gather(
                        rows_v, [jv, lvec, jnp.full((LANES,), c, jnp.int32)])
                    pacc[c] = pacc[c] + w * pc
            ssum = jnp.zeros((LANES,), jnp.float32)
            for c in range(N_PARTS):
                ssum = ssum + 1.0 / (1.0 + jnp.exp(-pacc[c]))
            s_v[pl.ds(cb * CH + g * LANES, LANES)] = ssum * macc * (1.0 / C_FEAT)
            return carry2

        lax.fori_loop(0, CH // LANES, group_body, 0)
        return carry

    lax.fori_loop(0, NCH, chunk_body, 0)
    pltpu.sync_copy(s_v, s_h.at[pl.ds(wid * PPT, PPT)])


_SC_PARAMS = pltpu.CompilerParams(
    needs_layout_passes=False, use_tc_tiling_on_sc=False)


def _sc_main(table, gidx, wts):
    kfn = pl.kernel(
        _sc_main_body,
        out_type=jax.ShapeDtypeStruct((N,), jnp.float32),
        mesh=plsc.VectorSubcoreMesh(core_axis_name="c", subcore_axis_name="s"),
        compiler_params=_SC_PARAMS,
        scratch_types=[
            pltpu.VMEM((NCORNER, CH), jnp.int32),
            pltpu.VMEM((NCORNER, CH), jnp.float32),
            pltpu.VMEM((NCORNER, CH, D), jnp.float32),
            pltpu.VMEM((PPT,), jnp.float32),
            pltpu.SemaphoreType.DMA,
        ],
    )
    return kfn(table, gidx, wts)


_RB = 128


def _cumsum_body(x_ref, out_ref, carry_ref):
    pid = pl.program_id(0)

    @pl.when(pid == 0)
    def _():
        carry_ref[0] = 0.0

    x = x_ref[...]
    r = lax.broadcasted_iota(jnp.int32, (128, 128), 0)
    cc = lax.broadcasted_iota(jnp.int32, (128, 128), 1)
    tri_incl = jnp.where(r <= cc, 1.0, 0.0)
    tri_strict = jnp.where(cc < r, 1.0, 0.0)
    rowcs = lax.dot_general(x, tri_incl, (((1,), (0,)), ((), ())),
                            preferred_element_type=jnp.float32,
                            precision=lax.Precision.HIGHEST)
    rowtot = jnp.sum(x, axis=1, keepdims=True)
    rowoff = lax.dot_general(tri_strict[:_RB, :_RB], rowtot,
                             (((1,), (0,)), ((), ())),
                             preferred_element_type=jnp.float32,
                             precision=lax.Precision.HIGHEST)
    out_ref[...] = rowcs + rowoff + carry_ref[0]
    carry_ref[0] = carry_ref[0] + jnp.sum(rowtot)


def _cumsum(s):
    x = s.reshape(N // 128, 128)
    out = pl.pallas_call(
        _cumsum_body,
        grid=(N // 128 // _RB,),
        in_specs=[pl.BlockSpec((_RB, 128), lambda p: (p, 0))],
        out_specs=pl.BlockSpec((_RB, 128), lambda p: (p, 0)),
        out_shape=jax.ShapeDtypeStruct((N // 128, 128), jnp.float32),
        scratch_shapes=[pltpu.SMEM((1,), jnp.float32)],
    )(x)
    return out.reshape(N)


def _sc_diff_body(e_h, idx_h, prev_h, out_h,
                  idxv, prevv, idxm, prevm, ev, pv, outv, sem):
    wid = lax.axis_index("s") * NC + lax.axis_index("c")
    base = wid * KT
    pltpu.sync_copy(idx_h.at[pl.ds(base, KT)], idxv)
    pltpu.sync_copy(prev_h.at[pl.ds(base, KT)], prevv)

    def clamp_body(t, carry):
        sl = pl.ds(t * LANES, LANES)
        idxm[sl] = jnp.maximum(idxv[sl] - 1, 0)
        prevm[sl] = jnp.maximum(prevv[sl] - 1, 0)
        return carry

    lax.fori_loop(0, KT // LANES, clamp_body, 0)

    def gather_body(j, carry):
        sl = pl.ds(j * 128, 128)
        pltpu.async_copy(e_h.at[idxm.at[sl]], ev.at[sl], sem).wait()
        pltpu.async_copy(e_h.at[prevm.at[sl]], pv.at[sl], sem).wait()
        return carry

    lax.fori_loop(0, KR, gather_body, 0)

    def diff_body(t, carry):
        sl = pl.ds(t * LANES, LANES)
        iv = idxv[sl]
        pvv = prevv[sl]
        zero = jnp.zeros((LANES,), jnp.float32)
        e = jnp.where(iv == 0, zero, ev[sl])
        p = jnp.where(pvv == 0, zero, pv[sl])
        outv[sl] = e - p
        return carry

    lax.fori_loop(0, KT // LANES, diff_body, 0)
    pltpu.sync_copy(outv, out_h.at[pl.ds(base, KT)])


def _sc_diff(e, idx_flat, prev):
    kfn = pl.kernel(
        _sc_diff_body,
        out_type=jax.ShapeDtypeStruct((K_OUT,), jnp.float32),
        mesh=plsc.VectorSubcoreMesh(core_axis_name="c", subcore_axis_name="s"),
        compiler_params=_SC_PARAMS,
        scratch_types=[
            pltpu.VMEM((KT,), jnp.int32),
            pltpu.VMEM((KT,), jnp.int32),
            pltpu.VMEM((KT,), jnp.int32),
            pltpu.VMEM((KT,), jnp.int32),
            pltpu.VMEM((KT,), jnp.float32),
            pltpu.VMEM((KT,), jnp.float32),
            pltpu.VMEM((KT,), jnp.float32),
            pltpu.SemaphoreType.DMA,
        ],
    )
    return kfn(e, idx_flat, prev)


def kernel(triplane, triplane_coord_3d, index):
    table = _pack_table(triplane)
    coords_t = jnp.zeros((8, N), jnp.float32).at[:3].set(
        jnp.transpose(triplane_coord_3d, (1, 0)))
    bounds = index[:, -1].reshape(1, B).astype(jnp.int32)
    gidx, wts = _prep_coords(coords_t, bounds)
    s = _sc_main(table, gidx, wts)
    e = _cumsum(s)
    idx_flat = index.reshape(-1).astype(jnp.int32)
    prev = jnp.concatenate([jnp.zeros((1,), idx_flat.dtype), idx_flat[:-1]])
    out = _sc_diff(e, idx_flat, prev)
    return out.reshape(B, NPTS)


# speedup vs baseline: 12.2532x; 3.7252x over previous
import functools

import jax
import jax.numpy as jnp
from jax import lax
from jax.experimental import pallas as pl
from jax.experimental.pallas import tpu as pltpu
from jax.experimental.pallas import tpu_sc as plsc

B = 4
C_FEAT = 32
N_PARTS = 16
C = C_FEAT + N_PARTS
H = 512
W = 512
N = 262144
NPTS = 32768
D = 32
NCORNER = 12

NC = 2
NS = 16
NW = NC * NS
LANES = 16
PPT = N // NW
CH = 128
NCH = PPT // CH
K_OUT = B * NPTS
KT = K_OUT // NW
KR = KT // 128


_HB = 16


_PPR = 128 // D
_OB = _HB * W // _PPR


def _table_body(tp_ref, out_ref):
    x = tp_ref[0]
    rr = lax.broadcasted_iota(jnp.int32, (_PPR * C, 128), 0)
    cc = lax.broadcasted_iota(jnp.int32, (_PPR * C, 128), 1)
    k = rr % C
    c0 = cc % D
    same_blk = (rr // C) == (cc // D)
    proj4 = jnp.where(
        same_blk & (((k == c0 + C_FEAT) & (c0 < N_PARTS))
                    | ((c0 >= N_PARTS) & (k < C_FEAT))), 1.0, 0.0)
    for hb in range(_HB):
        xs = x[:, hb, :]
        x4 = jnp.concatenate(
            [xs[:, p * 128:(p + 1) * 128] for p in range(_PPR)], axis=0)
        out_ref[pl.ds(hb * (W // _PPR), W // _PPR), :] = lax.dot_general(
            x4, proj4, (((0,), (0,)), ((), ())),
            preferred_element_type=jnp.float32)


def _pack_table(triplane):
    out = pl.pallas_call(
        _table_body,
        grid=(B, H // _HB),
        in_specs=[pl.BlockSpec((1, C, _HB, W), lambda b, h: (b, 0, h, 0))],
        out_specs=pl.BlockSpec((_OB, 128), lambda b, h: (b * (H // _HB) + h, 0)),
        out_shape=jax.ShapeDtypeStruct((B * H * W // _PPR, 128), jnp.float32),
    )(triplane)
    return out.reshape(B * H * W, D)


_PB = 2048
_GB = _PB // CH


def _coord_body(ct_ref, bounds_ref, gidx_ref, wts_ref):
    pid = pl.program_id(0)
    shape = (_GB, CH)
    i = (pid * _PB
         + lax.broadcasted_iota(jnp.int32, shape, 0) * CH
         + lax.broadcasted_iota(jnp.int32, shape, 1))
    bid = jnp.zeros(shape, jnp.int32)
    for j in range(B):
        bid = bid + (i >= bounds_ref[0, j]).astype(jnp.int32)
    bid = jnp.minimum(bid, B - 1)
    base = bid * (H * W)
    x = ct_ref[0].reshape(shape)
    y = ct_ref[1].reshape(shape)
    z = ct_ref[2].reshape(shape)
    px = jnp.clip((x + 1.0) * (0.5 * (W - 1.0)), 0.0, W - 1.0)
    py = jnp.clip((y + 1.0) * (0.5 * (H - 1.0)), 0.0, H - 1.0)
    pz = jnp.clip((z + 1.0) * (0.5 * (H - 1.0)), 0.0, H - 1.0)
    gs, ws = [], []
    for (u, v) in ((px, py), (px, pz), (py, pz)):
        bx = jnp.clip(jnp.floor(u), 0.0, W - 2.0)
        by = jnp.clip(jnp.floor(v), 0.0, H - 2.0)
        wx = u - bx
        wy = v - by
        i00 = base + by.astype(jnp.int32) * W + bx.astype(jnp.int32)
        for (dy, dx, wgt) in ((0, 0, (1.0 - wx) * (1.0 - wy)),
                              (0, 1, wx * (1.0 - wy)),
                              (1, 0, (1.0 - wx) * wy),
                              (1, 1, wx * wy)):
            g = i00 + (dy * W + dx)
            wq = g & (W - 1)
            g = (g - wq) + ((wq & 127) << 2) + (wq >> 7)
            gs.append(g)
            ws.append(wgt)
    gidx_ref[...] = jnp.stack(gs, axis=1).reshape(_GB * NCORNER, CH)
    wts_ref[...] = jnp.stack(ws, axis=1).reshape(_GB * NCORNER, CH)


def _prep_coords(coords_t, bounds):
    gidx, wts = pl.pallas_call(
        _coord_body,
        grid=(N // _PB,),
        in_specs=[
            pl.BlockSpec((8, _PB), lambda p: (0, p)),
            pl.BlockSpec(memory_space=pltpu.SMEM),
        ],
        out_specs=[
            pl.BlockSpec((_GB * NCORNER, CH), lambda p: (p, 0)),
            pl.BlockSpec((_GB * NCORNER, CH), lambda p: (p, 0)),
        ],
        out_shape=[
            jax.ShapeDtypeStruct((N // CH * NCORNER, CH), jnp.int32),
            jax.ShapeDtypeStruct((N // CH * NCORNER, CH), jnp.float32),
        ],
    )(coords_t, bounds)
    return (gidx.reshape(N // CH, NCORNER, CH),
            wts.reshape(N // CH, NCORNER, CH))


_MAXC = N // CH - 1


def _sc_main_body(table_h, gidx_h, wts_h, s_h, tot_h, idx_v, wts_v, rows_v,
                  s_v, tot_v, sem_r0, sem_r1, sem_i0, sem_i1):
    wid = lax.axis_index("s") * NC + lax.axis_index("c")
    chunk0 = wid * NCH
    sem_r = (sem_r0, sem_r1)
    sem_i = (sem_i0, sem_i1)

    def rows_slot(buf, j):
        return rows_v.at[pl.ds((buf * NCORNER + j) * CH, CH)]

    def issue_rows(buf, sem):
        for j in range(NCORNER):
            pltpu.async_copy(table_h.at[idx_v.at[buf, j]], rows_slot(buf, j), sem)

    def wait_rows(buf, sem):
        for j in range(NCORNER):
            pltpu.make_async_copy(
                table_h.at[idx_v.at[buf, j]], rows_slot(buf, j), sem).wait()

    def issue_idx(buf, gcb, sem):
        gcb = jnp.minimum(gcb, _MAXC)
        pltpu.async_copy(gidx_h.at[gcb], idx_v.at[buf], sem)
        pltpu.async_copy(wts_h.at[gcb], wts_v.at[buf], sem)

    def wait_idx(buf, gcb, sem):
        gcb = jnp.minimum(gcb, _MAXC)
        pltpu.make_async_copy(gidx_h.at[gcb], idx_v.at[buf], sem).wait()
        pltpu.make_async_copy(wts_h.at[gcb], wts_v.at[buf], sem).wait()

    def compute(buf, cb, carry):
        iot = lax.iota(jnp.int32, LANES)
        crot = [(iot + k) & (N_PARTS - 1) for k in range(N_PARTS)]
        cfs = iot + N_PARTS

        def group_body(g, carry2):
            lvec = lax.iota(jnp.int32, LANES) + g * LANES
            macc = jnp.zeros((LANES,), jnp.float32)
            pacc = [jnp.zeros((LANES,), jnp.float32) for _ in range(N_PARTS)]
            for j in range(NCORNER):
                w = wts_v[buf, j, pl.ds(g * LANES, LANES)]
                rvec = lvec + (buf * NCORNER + j) * CH
                fs = plsc.load_gather(rows_v, [rvec, cfs])
                macc = macc + w * fs
                for c in range(N_PARTS):
                    pc = plsc.load_gather(rows_v, [rvec, crot[c]])
                    pacc[c] = pacc[c] + w * pc
            ssum = jnp.zeros((LANES,), jnp.float32)
            for c in range(N_PARTS):
                ssum = ssum + 1.0 / (1.0 + jnp.exp(-pacc[c]))
            sv = ssum * macc * (1.0 / C_FEAT)
            s_v[pl.ds(cb * CH + g * LANES, LANES)] = plsc.cumsum(sv) + carry2
            return carry2 + jnp.sum(sv)

        return lax.fori_loop(0, CH // LANES, group_body, carry)

    issue_idx(0, chunk0, sem_i[0])
    wait_idx(0, chunk0, sem_i[0])
    issue_rows(0, sem_r[0])
    issue_idx(1, chunk0 + 1, sem_i[1])

    def pair_body(t, carry):
        for phase in range(2):
            buf = phase
            nbuf = 1 - phase
            cb = 2 * t + phase
            wait_rows(buf, sem_r[buf])
            wait_idx(nbuf, chunk0 + cb + 1, sem_i[nbuf])
            issue_rows(nbuf, sem_r[nbuf])
            carry = compute(buf, cb, carry)
            issue_idx(buf, chunk0 + cb + 2, sem_i[buf])
        return carry

    total = lax.fori_loop(0, NCH // 2, pair_body, jnp.float32(0.0))
    wait_rows(0, sem_r[0])
    wait_idx(1, chunk0 + NCH + 1, sem_i[1])
    pltpu.sync_copy(s_v, s_h.at[pl.ds(wid * PPT, PPT)])
    tot_v[...] = lax.broadcast_in_dim(total, (LANES,), ())
    pltpu.sync_copy(tot_v, tot_h.at[wid])


_SC_PARAMS = pltpu.CompilerParams(
    needs_layout_passes=False, use_tc_tiling_on_sc=False)


def _sc_main(table, gidx, wts):
    kfn = pl.kernel(
        _sc_main_body,
        out_type=[jax.ShapeDtypeStruct((N,), jnp.float32),
                  jax.ShapeDtypeStruct((NW, LANES), jnp.float32)],
        mesh=plsc.VectorSubcoreMesh(core_axis_name="c", subcore_axis_name="s"),
        compiler_params=_SC_PARAMS,
        scratch_types=[
            pltpu.VMEM((2, NCORNER, CH), jnp.int32),
            pltpu.VMEM((2, NCORNER, CH), jnp.float32),
            pltpu.VMEM((2 * NCORNER * CH, D), jnp.float32),
            pltpu.VMEM((PPT,), jnp.float32),
            pltpu.VMEM((LANES,), jnp.float32),
            pltpu.SemaphoreType.DMA,
            pltpu.SemaphoreType.DMA,
            pltpu.SemaphoreType.DMA,
            pltpu.SemaphoreType.DMA,
        ],
    )
    return kfn(table, gidx, wts)


def _sc_diff_body(e_h, tot_h, idx_h, prev_h, out_h,
                  idxv, prevv, idxm, prevm, ev, pv, outv, totv, offv, sem):
    wid = lax.axis_index("s") * NC + lax.axis_index("c")
    base = wid * KT
    pltpu.sync_copy(idx_h.at[pl.ds(base, KT)], idxv)
    pltpu.sync_copy(prev_h.at[pl.ds(base, KT)], prevv)
    pltpu.sync_copy(tot_h, totv)

    iot = lax.iota(jnp.int32, LANES)
    zc = jnp.zeros((LANES,), jnp.int32)
    t0 = plsc.load_gather(totv, [iot, zc])
    t1 = plsc.load_gather(totv, [iot + LANES, zc])
    c0 = plsc.cumsum(t0)
    c1 = plsc.cumsum(t1) + jnp.sum(t0)
    offv[pl.ds(0, LANES)] = c0 - t0
    offv[pl.ds(LANES, LANES)] = c1 - t1

    def clamp_body(t, carry):
        sl = pl.ds(t * LANES, LANES)
        idxm[sl] = jnp.maximum(idxv[sl] - 1, 0)
        prevm[sl] = jnp.maximum(prevv[sl] - 1, 0)
        return carry

    lax.fori_loop(0, KT // LANES, clamp_body, 0)

    def gather_issue(j, carry):
        sl = pl.ds(j * 128, 128)
        pltpu.async_copy(e_h.at[idxm.at[sl]], ev.at[sl], sem)
        pltpu.async_copy(e_h.at[prevm.at[sl]], pv.at[sl], sem)
        return carry

    lax.fori_loop(0, KR, gather_issue, 0)

    def gather_drain(j, carry):
        sl = pl.ds(j * 128, 128)
        pltpu.make_async_copy(e_h.at[idxm.at[sl]], ev.at[sl], sem).wait()
        pltpu.make_async_copy(e_h.at[prevm.at[sl]], pv.at[sl], sem).wait()
        return carry

    lax.fori_loop(0, KR, gather_drain, 0)

    def diff_body(t, carry):
        sl = pl.ds(t * LANES, LANES)
        iv = idxv[sl]
        pvv = prevv[sl]
        zero = jnp.zeros((LANES,), jnp.float32)
        oi = plsc.load_gather(offv, [idxm[sl] >> 13])
        op = plsc.load_gather(offv, [prevm[sl] >> 13])
        e = jnp.where(iv == 0, zero, ev[sl] + oi)
        p = jnp.where(pvv == 0, zero, pv[sl] + op)
        outv[sl] = e - p
        return carry

    lax.fori_loop(0, KT // LANES, diff_body, 0)
    pltpu.sync_copy(outv, out_h.at[pl.ds(base, KT)])


def _sc_diff(e, totals, idx_flat, prev):
    kfn = pl.kernel(
        _sc_diff_body,
        out_type=jax.ShapeDtypeStruct((K_OUT,), jnp.float32),
        mesh=plsc.VectorSubcoreMesh(core_axis_name="c", subcore_axis_name="s"),
        compiler_params=_SC_PARAMS,
        scratch_types=[
            pltpu.VMEM((KT,), jnp.int32),
            pltpu.VMEM((KT,), jnp.int32),
            pltpu.VMEM((KT,), jnp.int32),
            pltpu.VMEM((KT,), jnp.int32),
            pltpu.VMEM((KT,), jnp.float32),
            pltpu.VMEM((KT,), jnp.float32),
            pltpu.VMEM((KT,), jnp.float32),
            pltpu.VMEM((NW, LANES), jnp.float32),
            pltpu.VMEM((NW,), jnp.float32),
            pltpu.SemaphoreType.DMA,
        ],
    )
    return kfn(e, totals, idx_flat, prev)


def kernel(triplane, triplane_coord_3d, index):
    table = _pack_table(triplane)
    coords_t = jnp.zeros((8, N), jnp.float32).at[:3].set(
        jnp.transpose(triplane_coord_3d, (1, 0)))
    bounds = index[:, -1].reshape(1, B).astype(jnp.int32)
    gidx, wts = _prep_coords(coords_t, bounds)
    lcs, totals = _sc_main(table, gidx, wts)
    idx_flat = index.reshape(-1).astype(jnp.int32)
    prev = jnp.concatenate([jnp.zeros((1,), idx_flat.dtype), idx_flat[:-1]])
    out = _sc_diff(lcs, totals, idx_flat, prev)
    return out.reshape(B, NPTS)
